# Initial kernel scaffold; baseline (speedup 1.0000x reference)
import jax
import jax.numpy as jnp
from jax.experimental import pallas as pl


def kernel(indices, X_matrix, column_matrix):
    raise NotImplementedError("write your pallas kernel here")


# speedup vs baseline: 3.4194x; 3.4194x over previous
import functools

import jax
import jax.numpy as jnp
from jax import lax
from jax.experimental import pallas as pl
from jax.experimental.pallas import tpu as pltpu
from jax.experimental.pallas import tpu_sc as plsc

VOCAB = 100000
EMB = 64
SEQ = 50
BATCH = 4096

NC = 2
NS = 16
NW = NC * NS

ROWS = BATCH * SEQ
RPW = ROWS // NW
CH = 2 * SEQ
NCH = RPW // CH

_mesh = plsc.VectorSubcoreMesh(core_axis_name="c", subcore_axis_name="s")


@functools.partial(
    pl.kernel,
    mesh=_mesh,
    compiler_params=pltpu.CompilerParams(use_tc_tiling_on_sc=False),
    out_type=jax.ShapeDtypeStruct((2 * ROWS, EMB), jnp.float32),
    scratch_types=[
        pltpu.VMEM((NCH, CH), jnp.int32),
        pltpu.VMEM((NCH, CH), jnp.int32),
        pltpu.VMEM((NCH, CH), jnp.int32),
        pltpu.VMEM((CH, EMB), jnp.float32),
        pltpu.VMEM((CH, EMB), jnp.float32),
        pltpu.SemaphoreType.DMA,
        pltpu.SemaphoreType.DMA,
        pltpu.SemaphoreType.DMA,
    ],
)
def _emb_kernel(idx_hbm, x_hbm, col_hbm, ev_hbm, od_hbm, out_hbm,
                idx_v, ev_v, od_v, col_v, xbuf, gsem, ssem, csem):
    wid = lax.axis_index("s") * NC + lax.axis_index("c")
    base = wid * NCH
    pltpu.sync_copy(idx_hbm.at[pl.ds(base, NCH)], idx_v)
    pltpu.sync_copy(ev_hbm.at[pl.ds(base, NCH)], ev_v)
    pltpu.sync_copy(od_hbm.at[pl.ds(base, NCH)], od_v)
    pltpu.sync_copy(col_hbm, col_v.at[pl.ds(0, SEQ)])
    pltpu.sync_copy(col_hbm, col_v.at[pl.ds(SEQ, SEQ)])

    def body(j, carry):
        pltpu.async_copy(x_hbm.at[idx_v.at[j]], xbuf, gsem).wait()
        pltpu.async_copy(xbuf, out_hbm.at[ev_v.at[j]], ssem).wait()
        pltpu.async_copy(col_v, out_hbm.at[od_v.at[j]], csem).wait()
        return carry

    lax.fori_loop(0, NCH, body, 0)


def kernel(indices, X_matrix, column_matrix):
    idx2d = indices.astype(jnp.int32).reshape(ROWS // CH, CH)
    r = jnp.arange(ROWS, dtype=jnp.int32).reshape(ROWS // CH, CH)
    ev = r * 2
    od = ev + 1
    out = _emb_kernel(idx2d, X_matrix, column_matrix, ev, od)
    return out.reshape(BATCH, SEQ, 2 * EMB)


# speedup vs baseline: 3.8573x; 1.1281x over previous
import functools

import jax
import jax.numpy as jnp
from jax import lax
from jax.experimental import pallas as pl
from jax.experimental.pallas import tpu as pltpu
from jax.experimental.pallas import tpu_sc as plsc

VOCAB = 100000
EMB = 64
SEQ = 50
BATCH = 4096

NC = 2
NS = 16
NW = NC * NS

ROWS = BATCH * SEQ
RPW = ROWS // NW
CH = 128
NCH = RPW // CH
CREP = CH + SEQ

_mesh = plsc.VectorSubcoreMesh(core_axis_name="c", subcore_axis_name="s")


@functools.partial(
    pl.kernel,
    mesh=_mesh,
    compiler_params=pltpu.CompilerParams(use_tc_tiling_on_sc=False),
    out_type=jax.ShapeDtypeStruct((2 * ROWS, EMB), jnp.float32),
    scratch_types=[
        pltpu.VMEM((RPW,), jnp.int32),
        pltpu.VMEM((NCH, CH), jnp.int32),
        pltpu.VMEM((NCH, CH), jnp.int32),
        pltpu.VMEM((CREP, EMB), jnp.float32),
        pltpu.VMEM((CH, EMB), jnp.float32),
        pltpu.VMEM((CH, EMB), jnp.float32),
        pltpu.SemaphoreType.DMA,
        pltpu.SemaphoreType.DMA,
        pltpu.SemaphoreType.DMA,
        pltpu.SemaphoreType.DMA,
        pltpu.SemaphoreType.DMA,
        pltpu.SemaphoreType.DMA,
    ],
)
def _emb_kernel(idx_hbm, x_hbm, col_hbm, out_hbm,
                idx_v, ev_v, od_v, col_v, xb0, xb1,
                g0, g1, s0, s1, c0, c1):
    wid = lax.axis_index("s") * NC + lax.axis_index("c")
    wbase = wid * RPW
    pltpu.sync_copy(idx_hbm.at[pl.ds(wbase, RPW)], idx_v)

    xbufs = (xb0, xb1)
    gsems = (g0, g1)
    ssems = (s0, s1)
    csems = (c0, c1)

    def start_gather(j, b):
        pltpu.async_copy(x_hbm.at[idx_v.at[pl.ds(j * CH, CH)]], xbufs[b],
                         gsems[b])

    start_gather(0, 0)
    start_gather(1, 1)

    lanes = lax.iota(jnp.int32, 16)

    def build(j, carry):
        base = 2 * (wbase + j * CH)
        for k in range(CH // 16):
            v = base + 2 * (k * 16) + 2 * lanes
            ev_v[j, pl.ds(k * 16, 16)] = v
            od_v[j, pl.ds(k * 16, 16)] = v + 1
        return carry

    lax.fori_loop(0, NCH, build, 0)

    for t in range(CREP // SEQ):
        pltpu.sync_copy(col_hbm, col_v.at[pl.ds(t * SEQ, SEQ)])
    rem = CREP % SEQ
    if rem:
        pltpu.sync_copy(col_hbm.at[pl.ds(0, rem)],
                        col_v.at[pl.ds((CREP // SEQ) * SEQ, rem)])

    def body(i, carry):
        for b in range(2):
            j = 2 * i + b
            pltpu.make_async_copy(x_hbm.at[idx_v.at[pl.ds(j * CH, CH)]],
                                  xbufs[b], gsems[b]).wait()
            pltpu.async_copy(xbufs[b], out_hbm.at[ev_v.at[j]], ssems[b])
            off = lax.rem(j * CH, SEQ)
            pltpu.async_copy(col_v.at[pl.ds(off, CH)],
                             out_hbm.at[od_v.at[j]], csems[b])
            pltpu.make_async_copy(xbufs[b], out_hbm.at[ev_v.at[j]],
                                  ssems[b]).wait()
            pltpu.make_async_copy(col_v.at[pl.ds(off, CH)],
                                  out_hbm.at[od_v.at[j]], csems[b]).wait()

            @pl.when(j + 2 < NCH)
            def _():
                start_gather(j + 2, b)

        return carry

    lax.fori_loop(0, NCH // 2, body, 0)


def kernel(indices, X_matrix, column_matrix):
    idx_flat = indices.astype(jnp.int32).reshape(ROWS)
    out = _emb_kernel(idx_flat, X_matrix, column_matrix)
    return out.reshape(BATCH, SEQ, 2 * EMB)


# speedup vs baseline: 3.9466x; 1.0231x over previous
import functools

import jax
import jax.numpy as jnp
from jax import lax
from jax.experimental import pallas as pl
from jax.experimental.pallas import tpu as pltpu
from jax.experimental.pallas import tpu_sc as plsc

VOCAB = 100000
EMB = 64
SEQ = 50
BATCH = 4096

NC = 2
NS = 16
NW = NC * NS

ROWS = BATCH * SEQ
RPW = ROWS // NW
CH = 128
NCH = RPW // CH
CREP = CH + SEQ

_mesh = plsc.VectorSubcoreMesh(core_axis_name="c", subcore_axis_name="s")


@functools.partial(
    pl.kernel,
    mesh=_mesh,
    compiler_params=pltpu.CompilerParams(use_tc_tiling_on_sc=False),
    out_type=jax.ShapeDtypeStruct((2 * ROWS, EMB), jnp.float32),
    scratch_types=[
        pltpu.VMEM((NCH, CH), jnp.int32),
        pltpu.VMEM((NCH, CH), jnp.int32),
        pltpu.VMEM((NCH, CH), jnp.int32),
        pltpu.VMEM((CREP, EMB), jnp.float32),
        pltpu.VMEM((CH, EMB), jnp.float32),
        pltpu.VMEM((CH, EMB), jnp.float32),
        pltpu.SemaphoreType.DMA,
        pltpu.SemaphoreType.DMA,
        pltpu.SemaphoreType.DMA,
        pltpu.SemaphoreType.DMA,
        pltpu.SemaphoreType.DMA,
        pltpu.SemaphoreType.DMA,
    ],
)
def _emb_kernel(idx_hbm, x_hbm, col_hbm, out_hbm,
                idx_v, ev_v, od_v, col_v, xb0, xb1,
                g0, g1, s0, s1, c0, c1):
    wid = lax.axis_index("s") * NC + lax.axis_index("c")
    wbase = wid * RPW
    pltpu.sync_copy(idx_hbm.at[pl.ds(wid * NCH, NCH)], idx_v)

    xbufs = (xb0, xb1)
    gsems = (g0, g1)
    ssems = (s0, s1)
    csems = (c0, c1)

    def start_gather(j, b):
        pltpu.async_copy(x_hbm.at[idx_v.at[j]], xbufs[b], gsems[b])

    start_gather(0, 0)
    start_gather(1, 1)

    lanes = lax.iota(jnp.int32, 16)

    def build(j, carry):
        base = 2 * (wbase + j * CH)
        for k in range(CH // 16):
            v = base + 2 * (k * 16) + 2 * lanes
            ev_v[j, pl.ds(k * 16, 16)] = v
            od_v[j, pl.ds(k * 16, 16)] = v + 1
        return carry

    lax.fori_loop(0, NCH, build, 0)

    for t in range(CREP // SEQ):
        pltpu.sync_copy(col_hbm, col_v.at[pl.ds(t * SEQ, SEQ)])
    rem = CREP % SEQ
    if rem:
        pltpu.sync_copy(col_hbm.at[pl.ds(0, rem)],
                        col_v.at[pl.ds((CREP // SEQ) * SEQ, rem)])

    def body(i, carry):
        for b in range(2):
            j = 2 * i + b
            pltpu.make_async_copy(x_hbm.at[idx_v.at[j]],
                                  xbufs[b], gsems[b]).wait()
            pltpu.async_copy(xbufs[b], out_hbm.at[ev_v.at[j]], ssems[b])
            off = lax.rem(j * CH, SEQ)
            pltpu.async_copy(col_v.at[pl.ds(off, CH)],
                             out_hbm.at[od_v.at[j]], csems[b])
            pltpu.make_async_copy(xbufs[b], out_hbm.at[ev_v.at[j]],
                                  ssems[b]).wait()
            pltpu.make_async_copy(col_v.at[pl.ds(off, CH)],
                                  out_hbm.at[od_v.at[j]], csems[b]).wait()

            @pl.when(j + 2 < NCH)
            def _():
                start_gather(j + 2, b)

        return carry

    lax.fori_loop(0, NCH // 2, body, 0)


def kernel(indices, X_matrix, column_matrix):
    idx2d = (indices.astype(jnp.int32) * 2).reshape(ROWS // CH, CH)
    x2 = jnp.pad(X_matrix, ((0, 0), (0, EMB))).reshape(2 * VOCAB, EMB)
    out = _emb_kernel(idx2d, x2, column_matrix)
    return out.reshape(BATCH, SEQ, 2 * EMB)


# speedup vs baseline: 3.9746x; 1.0071x over previous
import functools

import jax
import jax.numpy as jnp
from jax import lax
from jax.experimental import pallas as pl
from jax.experimental.pallas import tpu as pltpu
from jax.experimental.pallas import tpu_sc as plsc

VOCAB = 100000
EMB = 64
SEQ = 50
BATCH = 4096

NC = 2
NS = 16
NW = NC * NS

ROWS = BATCH * SEQ
RPW = ROWS // NW
CH = 2 * SEQ
NCH = RPW // CH

_mesh = plsc.VectorSubcoreMesh(core_axis_name="c", subcore_axis_name="s")


@functools.partial(
    pl.kernel,
    mesh=_mesh,
    compiler_params=pltpu.CompilerParams(use_tc_tiling_on_sc=False),
    out_type=jax.ShapeDtypeStruct((ROWS, 2 * EMB), jnp.float32),
    scratch_types=[
        pltpu.VMEM((NCH, CH), jnp.int32),
        pltpu.VMEM((CH, EMB), jnp.float32),
        pltpu.VMEM((CH, EMB), jnp.float32),
        pltpu.VMEM((CH, EMB), jnp.float32),
        pltpu.SemaphoreType.DMA,
        pltpu.SemaphoreType.DMA,
        pltpu.SemaphoreType.DMA,
        pltpu.SemaphoreType.DMA,
        pltpu.SemaphoreType.DMA,
        pltpu.SemaphoreType.DMA,
    ],
)
def _emb_kernel(idx_hbm, x_hbm, col_hbm, out_hbm,
                idx_v, xb0, xb1, col_v, g0, g1, w0, w1, c0, c1):
    wid = lax.axis_index("s") * NC + lax.axis_index("c")
    wbase = wid * RPW
    pltpu.sync_copy(idx_hbm.at[pl.ds(wid * NCH, NCH)], idx_v)

    xbufs = (xb0, xb1)
    gsems = (g0, g1)
    wsems = (w0, w1)
    csems = (c0, c1)

    def start_gather(j, b):
        pltpu.async_copy(x_hbm.at[idx_v.at[j]], xbufs[b], gsems[b])

    start_gather(0, 0)
    start_gather(1, 1)

    pltpu.sync_copy(col_hbm, col_v.at[pl.ds(0, SEQ)])
    pltpu.sync_copy(col_hbm, col_v.at[pl.ds(SEQ, SEQ)])

    def body(i, carry):
        for b in range(2):
            j = 2 * i + b
            rows = pl.ds(wbase + j * CH, CH)
            pltpu.make_async_copy(x_hbm.at[idx_v.at[j]],
                                  xbufs[b], gsems[b]).wait()
            pltpu.async_copy(xbufs[b], out_hbm.at[rows, pl.ds(0, EMB)],
                             wsems[b])
            pltpu.async_copy(col_v, out_hbm.at[rows, pl.ds(EMB, EMB)],
                             csems[b])

            @pl.when(j + 2 < NCH)
            def _():
                start_gather(j + 2, b)

            pltpu.make_async_copy(xbufs[b], out_hbm.at[rows, pl.ds(0, EMB)],
                                  wsems[b]).wait()
            pltpu.make_async_copy(col_v, out_hbm.at[rows, pl.ds(EMB, EMB)],
                                  csems[b]).wait()
        return carry

    lax.fori_loop(0, NCH // 2, body, 0)


def kernel(indices, X_matrix, column_matrix):
    idx2d = (indices.astype(jnp.int32) * 2).reshape(ROWS // CH, CH)
    x2 = jnp.pad(X_matrix, ((0, 0), (0, EMB))).reshape(2 * VOCAB, EMB)
    out = _emb_kernel(idx2d, x2, column_matrix)
    return out.reshape(BATCH, SEQ, 2 * EMB)


# speedup vs baseline: 9.1453x; 2.3009x over previous
import functools

import jax
import jax.numpy as jnp
from jax import lax
from jax.experimental import pallas as pl
from jax.experimental.pallas import tpu as pltpu
from jax.experimental.pallas import tpu_sc as plsc

VOCAB = 100000
EMB = 64
SEQ = 50
BATCH = 4096

NC = 2
NS = 16
NW = NC * NS

ROWS = BATCH * SEQ
RPW = ROWS // NW
CH = 128
NCH = RPW // CH
CREP = CH + SEQ

NB = 6
LAG = 3

_mesh = plsc.VectorSubcoreMesh(core_axis_name="c", subcore_axis_name="s")

_SCRATCH = (
    [pltpu.VMEM((NCH, CH), jnp.int32)] * 3
    + [pltpu.VMEM((CREP, EMB), jnp.float32)]
    + [pltpu.VMEM((CH, EMB), jnp.float32)] * NB
    + [pltpu.SemaphoreType.DMA] * (3 * NB)
)


def _emb_body(idx_hbm, x_hbm, col_hbm, out_hbm, idx_v, ev_v, od_v, col_v,
              *bufs_and_sems):
    xbufs = bufs_and_sems[:NB]
    gsems = bufs_and_sems[NB:2 * NB]
    ssems = bufs_and_sems[2 * NB:3 * NB]
    csems = bufs_and_sems[3 * NB:4 * NB]

    wid = lax.axis_index("s") * NC + lax.axis_index("c")
    wbase = wid * RPW
    pltpu.sync_copy(idx_hbm.at[pl.ds(wid * NCH, NCH)], idx_v)

    def start_gather(j, b):
        pltpu.async_copy(x_hbm.at[idx_v.at[j]], xbufs[b], gsems[b])

    def wait_gather(j, b):
        pltpu.make_async_copy(x_hbm.at[idx_v.at[j]], xbufs[b],
                              gsems[b]).wait()

    def col_src(j):
        return col_v.at[pl.ds(lax.rem(j * CH, SEQ), CH)]

    def start_scats(j, b):
        pltpu.async_copy(xbufs[b], out_hbm.at[ev_v.at[j]], ssems[b])
        pltpu.async_copy(col_src(j), out_hbm.at[od_v.at[j]], csems[b])

    def wait_scats(j, b):
        pltpu.make_async_copy(xbufs[b], out_hbm.at[ev_v.at[j]],
                              ssems[b]).wait()
        pltpu.make_async_copy(col_src(j), out_hbm.at[od_v.at[j]],
                              csems[b]).wait()

    lanes = lax.iota(jnp.int32, 16)

    def build(j, carry):
        base = wbase + j * CH
        for k in range(CH // 16):
            r = base + k * 16 + lanes
            p = lax.rem(r, SEQ) * BATCH + lax.div(r, SEQ)
            ev_v[j, pl.ds(k * 16, 16)] = 2 * p
            od_v[j, pl.ds(k * 16, 16)] = 2 * p + 1
        return carry

    lax.fori_loop(0, NCH, build, 0)

    for t in range(CREP // SEQ):
        pltpu.sync_copy(col_hbm, col_v.at[pl.ds(t * SEQ, SEQ)])
    rem = CREP % SEQ
    if rem:
        pltpu.sync_copy(col_hbm.at[pl.ds(0, rem)],
                        col_v.at[pl.ds((CREP // SEQ) * SEQ, rem)])

    NSTEP = NCH + LAG
    NGRP = (NSTEP + NB - 1) // NB

    def group(i, carry):
        for b in range(NB):
            j = i * NB + b

            @pl.when(jnp.logical_and(j >= NB, j < NCH))
            def _():
                wait_scats(j - NB, b)

            @pl.when(j < NCH)
            def _():
                start_gather(j, b)

            s = j - LAG
            sb = (b - LAG) % NB

            @pl.when(jnp.logical_and(s >= 0, s < NCH))
            def _():
                wait_gather(s, sb)
                start_scats(s, sb)

        return carry

    lax.fori_loop(0, NGRP, group, 0)

    for b in range(NB):
        j = NCH - NB + b
        wait_scats(j, j % NB)


_emb_kernel = functools.partial(
    pl.kernel,
    mesh=_mesh,
    compiler_params=pltpu.CompilerParams(use_tc_tiling_on_sc=False),
    out_type=jax.ShapeDtypeStruct((2 * ROWS, EMB), jnp.float32),
    scratch_types=_SCRATCH,
)(_emb_body)


def kernel(indices, X_matrix, column_matrix):
    idx2d = indices.astype(jnp.int32).reshape(ROWS // CH, CH)
    out = _emb_kernel(idx2d, X_matrix, column_matrix)
    return out.reshape(SEQ, BATCH, 2 * EMB).transpose(1, 0, 2)


# speedup vs baseline: 9.7514x; 1.0663x over previous
import functools

import jax
import jax.numpy as jnp
from jax import lax
from jax.experimental import pallas as pl
from jax.experimental.pallas import tpu as pltpu
from jax.experimental.pallas import tpu_sc as plsc

VOCAB = 100000
EMB = 64
SEQ = 50
BATCH = 4096

NC = 2
NS = 16
NW = NC * NS

ROWS = BATCH * SEQ
RPW = ROWS // NW
CH = 128
NCH = RPW // CH
CREP = CH + SEQ

NB = 6
LAG = 3

_mesh = plsc.VectorSubcoreMesh(core_axis_name="c", subcore_axis_name="s")

_SCRATCH = (
    [pltpu.VMEM((NCH, CH), jnp.int32)] * 3
    + [pltpu.VMEM((CREP, EMB), jnp.float32)]
    + [pltpu.VMEM((CH, EMB), jnp.float32)] * NB
    + [pltpu.SemaphoreType.DMA] * (3 * NB)
)


def _emb_body(idx_hbm, x_hbm, col_hbm, out_hbm, idx_v, ev_v, od_v, col_v,
              *bufs_and_sems):
    xbufs = bufs_and_sems[:NB]
    gsems = bufs_and_sems[NB:2 * NB]
    ssems = bufs_and_sems[2 * NB:3 * NB]
    csems = bufs_and_sems[3 * NB:4 * NB]

    wid = lax.axis_index("s") * NC + lax.axis_index("c")
    wbase = wid * RPW
    pltpu.sync_copy(idx_hbm.at[pl.ds(wid * NCH, NCH)], idx_v)

    def start_gather(j, b):
        pltpu.async_copy(x_hbm.at[idx_v.at[j]], xbufs[b], gsems[b])

    def wait_gather(j, b):
        pltpu.make_async_copy(x_hbm.at[idx_v.at[j]], xbufs[b],
                              gsems[b]).wait()

    def col_src(j):
        return col_v.at[pl.ds(lax.rem(j * CH, SEQ), CH)]

    def start_scats(j, b):
        pltpu.async_copy(xbufs[b], out_hbm.at[ev_v.at[j]], ssems[b])
        pltpu.async_copy(col_src(j), out_hbm.at[od_v.at[j]], csems[b])

    def wait_scats(j, b):
        pltpu.make_async_copy(xbufs[b], out_hbm.at[ev_v.at[j]],
                              ssems[b]).wait()
        pltpu.make_async_copy(col_src(j), out_hbm.at[od_v.at[j]],
                              csems[b]).wait()

    lanes = lax.iota(jnp.int32, 16)

    def build(j, carry):
        base = wbase + j * CH
        for k in range(CH // 16):
            r = base + k * 16 + lanes
            p = lax.rem(r, SEQ) * BATCH + lax.div(r, SEQ)
            ev_v[j, pl.ds(k * 16, 16)] = 2 * p
            od_v[j, pl.ds(k * 16, 16)] = 2 * p + 1
        return carry

    lax.fori_loop(0, NCH, build, 0)

    for t in range(CREP // SEQ):
        pltpu.sync_copy(col_hbm, col_v.at[pl.ds(t * SEQ, SEQ)])
    rem = CREP % SEQ
    if rem:
        pltpu.sync_copy(col_hbm.at[pl.ds(0, rem)],
                        col_v.at[pl.ds((CREP // SEQ) * SEQ, rem)])

    NSTEP = NCH + LAG
    NGRP = (NSTEP + NB - 1) // NB

    def group(i, carry):
        for b in range(NB):
            j = i * NB + b

            @pl.when(jnp.logical_and(j >= NB, j < NCH))
            def _():
                wait_scats(j - NB, b)

            @pl.when(j < NCH)
            def _():
                start_gather(j, b)

            s = j - LAG
            sb = (b - LAG) % NB

            @pl.when(jnp.logical_and(s >= 0, s < NCH))
            def _():
                wait_gather(s, sb)
                start_scats(s, sb)

        return carry

    lax.fori_loop(0, NGRP, group, 0)

    for b in range(NB):
        j = NCH - NB + b
        wait_scats(j, j % NB)


_emb_kernel = functools.partial(
    pl.kernel,
    mesh=_mesh,
    compiler_params=pltpu.CompilerParams(use_tc_tiling_on_sc=False),
    out_type=jax.ShapeDtypeStruct((2 * ROWS, EMB), jnp.float32),
    scratch_types=_SCRATCH,
)(_emb_body)


def kernel(indices, X_matrix, column_matrix):
    idx2d = (indices.astype(jnp.int32) * 2).reshape(ROWS // CH, CH)
    x2 = jnp.pad(X_matrix, ((0, 0), (0, EMB))).reshape(2 * VOCAB, EMB)
    out = _emb_kernel(idx2d, x2, column_matrix)
    return out.reshape(SEQ, BATCH, 2 * EMB).transpose(1, 0, 2)
